# baseline (speedup 1.0000x reference)
import functools

import jax
import jax.numpy as jnp
from jax import lax
from jax.experimental import pallas as pl
from jax.experimental.pallas import tpu as pltpu
from jax.experimental.pallas import tpu_sc as plsc

N_NODES = 10000
D_FEAT = 128
N_EDGES = 320000

NC = 2
NS = 16
NW = NC * NS
EPW = N_EDGES // NW
K = 80
NB = 5
CB = 25
N_PAD = 10240
RPT = N_PAD // NS


NBUF = 3


def _sc_scatter_body(x_hbm, src_hbm, dst_hbm, ew_hbm, acc_hbm,
                     acc_sh, src_v, dst_v, ew_v,
                     rows0, rows1, rows2,
                     g0, g1, g2, s0, s1, s2):
    rows = (rows0, rows1, rows2)
    gsem = (g0, g1, g2)
    ssem = (s0, s1, s2)
    c = lax.axis_index("c")
    s = lax.axis_index("s")
    gid = c * NS + s

    def _zero(i, carry):
        rows0[i // 8, pl.ds((i % 8) * 16, 16)] = jnp.zeros((16,), jnp.float32)
        return carry
    lax.fori_loop(0, K * 8, _zero, 0)
    for j in range(RPT // K):
        pltpu.sync_copy(rows0, acc_sh.at[pl.ds(s * RPT + j * K, K)])
    plsc.subcore_barrier()

    def _scale(buf, base, g, inner):
        w_win = ew_v[pl.ds(base + g * 16, 16)]
        for r16 in range(16):
            r = g * 16 + r16
            w16 = jnp.broadcast_to(w_win[r16], (16,))
            for cc in range(8):
                sl = pl.ds(cc * 16, 16)
                buf[r, sl] = buf[r, sl] * w16
        return inner

    def _block(b, carry):
        pltpu.sync_copy(src_hbm.at[gid, b], src_v)
        pltpu.sync_copy(dst_hbm.at[gid, b], dst_v)
        pltpu.sync_copy(ew_hbm.at[gid, b], ew_v)

        for i in range(NBUF):
            pltpu.async_copy(x_hbm.at[src_v.at[i]], rows[i], gsem[i])

        def _quad(q, c2):
            k0 = q * NBUF
            for i in range(NBUF):
                k = k0 + i
                pltpu.make_async_copy(x_hbm.at[src_v.at[k]], rows[i],
                                      gsem[i]).wait()
                lax.fori_loop(0, K // 16,
                              functools.partial(_scale, rows[i], k * K), 0)
                pltpu.async_copy(rows[i], acc_sh.at[dst_v.at[k]],
                                 ssem[i], add=True)
            for i in range(NBUF):
                k = k0 + i
                pltpu.make_async_copy(rows[i], acc_sh.at[dst_v.at[k]],
                                      ssem[i]).wait()
                kn = k0 + NBUF + i

                @pl.when(kn < CB)
                def _issue_next(kn=kn, i=i):
                    pltpu.async_copy(x_hbm.at[src_v.at[kn]], rows[i],
                                     gsem[i])
            return c2
        lax.fori_loop(0, CB // NBUF, _quad, 0)

        kt = (CB // NBUF) * NBUF
        pltpu.make_async_copy(x_hbm.at[src_v.at[kt]], rows0, g0).wait()
        lax.fori_loop(0, K // 16,
                      functools.partial(_scale, rows0, kt * K), 0)
        pltpu.sync_copy(rows0, acc_sh.at[dst_v.at[kt]], add=True)
        return carry
    lax.fori_loop(0, NB, _block, 0)

    plsc.subcore_barrier()
    pltpu.sync_copy(acc_sh.at[pl.ds(s * RPT, RPT)],
                    acc_hbm.at[c, pl.ds(s * RPT, RPT)])


_sc_scatter = functools.partial(
    pl.kernel,
    out_type=jax.ShapeDtypeStruct((NC, N_PAD, D_FEAT), jnp.float32),
    mesh=plsc.VectorSubcoreMesh(core_axis_name="c", subcore_axis_name="s"),
    scratch_types=[
        pltpu.VMEM_SHARED((N_PAD, D_FEAT), jnp.float32),
        pltpu.VMEM((CB, K), jnp.int32),
        pltpu.VMEM((CB, K), jnp.int32),
        pltpu.VMEM((CB * K,), jnp.float32),
        pltpu.VMEM((K, D_FEAT), jnp.float32),
        pltpu.VMEM((K, D_FEAT), jnp.float32),
        pltpu.VMEM((K, D_FEAT), jnp.float32),
        pltpu.SemaphoreType.DMA,
        pltpu.SemaphoreType.DMA,
        pltpu.SemaphoreType.DMA,
        pltpu.SemaphoreType.DMA,
        pltpu.SemaphoreType.DMA,
        pltpu.SemaphoreType.DMA,
    ],
)(_sc_scatter_body)


def _finish_body(acc_ref, o_ref):
    t = acc_ref[0] + acc_ref[1]
    t = jnp.maximum(t, 0.0)
    nrm = jnp.sqrt(jnp.sum(t * t, axis=1, keepdims=True))
    o_ref[...] = t / jnp.maximum(nrm, 1e-12)


_ROWS_PER_BLK = 1024


def _finish(acc):
    return pl.pallas_call(
        _finish_body,
        grid=(N_PAD // _ROWS_PER_BLK,),
        in_specs=[pl.BlockSpec((NC, _ROWS_PER_BLK, D_FEAT),
                               lambda i: (0, i, 0))],
        out_specs=pl.BlockSpec((_ROWS_PER_BLK, D_FEAT), lambda i: (i, 0)),
        out_shape=jax.ShapeDtypeStruct((N_PAD, D_FEAT), jnp.float32),
    )(acc)


def kernel(x, edge, edge_weight):
    src = edge[0].reshape(NW, NB, CB, K)
    dst = edge[2].reshape(NW, NB, CB, K)
    ew = edge_weight.reshape(NW, NB, CB * K)
    acc = _sc_scatter(x, src, dst, ew)
    return _finish(acc)[:N_NODES]
